# baseline (speedup 1.0000x reference)
import jax
import jax.numpy as jnp
from jax import lax
from jax.experimental import pallas as pl
from jax.experimental.pallas import tpu as pltpu
from jax.experimental.pallas import tpu_sc as plsc

P = 97
TOTAL = 32768
NSUB = 16
CHUNK = TOTAL // NSUB
NVEC = CHUNK // 16

_IN_BOUNDS = lax.GatherScatterMode.PROMISE_IN_BOUNDS


def _take16(v, idx):
    return v.at[idx].get(mode=_IN_BOUNDS)


def _mod_p(x):
    return jnp.where(x >= P, x - P, x)


def _subcore_id():
    return lax.axis_index("s")


def _sc_body(tokens_hbm, cu_hbm, res_hbm, counts_hbm, stage_hbm,
             chunk, cuv, outv, partial, red, cnt, sem):
    sid = _subcore_id()
    base = sid * CHUNK

    lh = jnp.where(sid == 0, TOTAL - 16, base - 16)
    rh = jnp.where(sid == NSUB - 1, 0, base + CHUNK)
    c0 = pltpu.make_async_copy(
        tokens_hbm.at[pl.ds(base, CHUNK)], chunk.at[pl.ds(16, CHUNK)], sem)
    c1 = pltpu.make_async_copy(
        tokens_hbm.at[pl.ds(lh, 16)], chunk.at[pl.ds(0, 16)], sem)
    c2 = pltpu.make_async_copy(
        tokens_hbm.at[pl.ds(rh, 16)], chunk.at[pl.ds(2048 + 16, 16)], sem)
    c3 = pltpu.make_async_copy(
        cu_hbm.at[pl.ds(0, 16)], cuv.at[pl.ds(0, 16)], sem)
    c0.start()
    c1.start()
    c2.start()
    c3.start()
    c0.wait()
    c1.wait()
    c2.wait()
    c3.wait()

    lane = lax.broadcasted_iota(jnp.int32, (16,), 0)
    cu_lo = cuv[pl.ds(0, 16)]
    cu_hi = jnp.where(lane == 15, TOTAL, cuv[pl.ds(1, 16)])
    zeros = jnp.zeros((16,), jnp.int32)
    big = jnp.int32(1 << 30)
    seg0 = jnp.sum(jnp.where(cu_hi <= base, 1, 0))
    nb0 = jnp.min(jnp.where(cu_hi > base, cu_hi, big))

    def body(k, carry):
        acc, acc2, seg, nb = carry
        off = 16 + k * 16
        t = chunk[pl.ds(off, 16)]
        tl = chunk[pl.ds(off - 1, 16)]
        tr = chunk[pl.ds(off + 1, 16)]

        l = _mod_p(tl)
        r = _mod_p(tr)
        add_r = _mod_p(l + r)
        d = l - r
        sub_r = jnp.where(d < 0, d + P, d)
        m = l * r
        q = jnp.right_shift(m * 676, 16)
        r0 = m - q * P
        mul_r = jnp.where(r0 < 0, r0 + P, r0)
        opi = t - P
        res = jnp.where(opi == 0, add_r, jnp.where(opi == 1, sub_r, mul_r))
        res = jnp.where(t < P, t, jnp.where(t >= P + 3, t, res))
        outv[pl.ds(k * 16, 16)] = res

        p0 = base + k * 16
        one = jnp.ones((16,), jnp.int32)
        packed = jnp.where(t < P, one, jnp.where(t < P + 3, one * 65536, 0))

        def fast(ops):
            acc, acc2, seg, nb = ops
            return acc, acc2 + packed, seg, nb

        def slow(ops):
            acc, acc2, seg, nb = ops
            acc = acc + jnp.where(lane == seg, jnp.sum(acc2), 0)
            lo = jnp.clip(cu_lo - p0, 0, 16)
            hi = jnp.clip(cu_hi - p0, 0, 16)
            ilo = jnp.maximum(lo - 1, 0)
            ihi = jnp.maximum(hi - 1, 0)
            cs = plsc.cumsum(packed)
            diff = jnp.where(hi > 0, _take16(cs, ihi), 0) - \
                jnp.where(lo > 0, _take16(cs, ilo), 0)
            seg2 = jnp.sum(jnp.where(cu_hi <= p0 + 16, 1, 0))
            nb2 = jnp.min(jnp.where(cu_hi > p0 + 16, cu_hi, big))
            return acc + diff, zeros, seg2, nb2

        return lax.cond(nb < p0 + 16, slow, fast, (acc, acc2, seg, nb))

    acc, acc2, seg, _ = lax.fori_loop(
        0, NVEC, body, (zeros, zeros, seg0, nb0), unroll=4)
    acc = acc + jnp.where(lane == seg, jnp.sum(acc2), 0)
    acc_o = jnp.bitwise_and(acc, 0xFFFF)
    acc_p = jnp.right_shift(acc, 16)

    cres = pltpu.make_async_copy(outv, res_hbm.at[pl.ds(base, CHUNK)], sem)
    cres.start()

    partial[0, :] = acc_o
    partial[1, :] = acc_p
    pltpu.sync_copy(partial, stage_hbm.at[sid])
    plsc.subcore_barrier()

    @pl.when(sid == 0)
    def _():
        pltpu.sync_copy(stage_hbm, red)
        ao = jnp.zeros((16,), jnp.int32)
        ap = jnp.zeros((16,), jnp.int32)
        for tix in range(NSUB):
            ao = ao + red[tix, 0, :]
            ap = ap + red[tix, 1, :]
        fo = ao.astype(jnp.float32)
        fp = ap.astype(jnp.float32)
        seg = lax.broadcasted_iota(jnp.int32, (16,), 0)
        plsc.store_scatter(cnt, [seg, zeros], fo)
        plsc.store_scatter(cnt, [seg, zeros + 1], fp)
        plsc.store_scatter(cnt, [seg, zeros + 2], fo + fp)
        pltpu.sync_copy(cnt, counts_hbm)

    cres.wait()


@jax.jit
def _run(tokens, cu_pad):
    mesh = plsc.VectorSubcoreMesh(
        core_axis_name="c", subcore_axis_name="s", num_cores=1)
    kern = pl.kernel(
        _sc_body,
        out_type=[
            jax.ShapeDtypeStruct((TOTAL,), jnp.int32),
            jax.ShapeDtypeStruct((16, 3), jnp.float32),
            jax.ShapeDtypeStruct((NSUB, 2, 16), jnp.int32),
        ],
        mesh=mesh,
        compiler_params=pltpu.CompilerParams(needs_layout_passes=False),
        scratch_types=[
            pltpu.VMEM((CHUNK + 32,), jnp.int32),
            pltpu.VMEM((24,), jnp.int32),
            pltpu.VMEM((CHUNK,), jnp.int32),
            pltpu.VMEM((2, 16), jnp.int32),
            pltpu.VMEM((NSUB, 2, 16), jnp.int32),
            pltpu.VMEM((16, 3), jnp.float32),
            pltpu.SemaphoreType.DMA,
        ],
    )
    return kern(tokens, cu_pad)


def kernel(tokens, cu_seqlens):
    res, counts, _ = _run(tokens, cu_seqlens)
    return res, counts


# speedup vs baseline: 1.0282x; 1.0282x over previous
import jax
import jax.numpy as jnp
from jax import lax
from jax.experimental import pallas as pl
from jax.experimental.pallas import tpu as pltpu
from jax.experimental.pallas import tpu_sc as plsc

P = 97
TOTAL = 32768
NSUB = 16
CHUNK = TOTAL // NSUB
NVEC = CHUNK // 16

_IN_BOUNDS = lax.GatherScatterMode.PROMISE_IN_BOUNDS


def _take16(v, idx):
    return v.at[idx].get(mode=_IN_BOUNDS)


def _mod_p(x):
    return jnp.where(x >= P, x - P, x)


def _subcore_id():
    return lax.axis_index("s")


def _sc_body(tokens_hbm, cu_hbm, res_hbm, counts_hbm, stage_hbm,
             chunk, cuv, outv, partial, red, cnt, sem):
    sid = _subcore_id()
    base = sid * CHUNK

    lh = jnp.where(sid == 0, TOTAL - 16, base - 16)
    rh = jnp.where(sid == NSUB - 1, 0, base + CHUNK)
    c0 = pltpu.make_async_copy(
        tokens_hbm.at[pl.ds(base, CHUNK)], chunk.at[pl.ds(16, CHUNK)], sem)
    c1 = pltpu.make_async_copy(
        tokens_hbm.at[pl.ds(lh, 16)], chunk.at[pl.ds(0, 16)], sem)
    c2 = pltpu.make_async_copy(
        tokens_hbm.at[pl.ds(rh, 16)], chunk.at[pl.ds(2048 + 16, 16)], sem)
    c3 = pltpu.make_async_copy(
        cu_hbm.at[pl.ds(0, 16)], cuv.at[pl.ds(0, 16)], sem)
    c0.start()
    c1.start()
    c2.start()
    c3.start()
    c0.wait()
    c1.wait()
    c2.wait()
    c3.wait()

    lane = lax.broadcasted_iota(jnp.int32, (16,), 0)
    cu_lo = cuv[pl.ds(0, 16)]
    cu_hi = jnp.where(lane == 15, TOTAL, cuv[pl.ds(1, 16)])
    zeros = jnp.zeros((16,), jnp.int32)
    big = jnp.int32(1 << 30)
    seg0 = jnp.sum(jnp.where(cu_hi <= base, 1, 0))
    nb0 = jnp.min(jnp.where(cu_hi > base, cu_hi, big))

    GRP = 8
    one = jnp.ones((16,), jnp.int32)

    def body(g, carry):
        acc, acc2, seg, nb = carry
        packs = []
        for j in range(GRP):
            k = g * GRP + j
            off = 16 + k * 16
            t = chunk[pl.ds(off, 16)]
            tl = chunk[pl.ds(off - 1, 16)]
            tr = chunk[pl.ds(off + 1, 16)]

            l = _mod_p(tl)
            r = _mod_p(tr)
            add_r = _mod_p(l + r)
            d = l - r
            sub_r = jnp.where(d < 0, d + P, d)
            m = l * r
            q = jnp.right_shift(m * 676, 16)
            r0 = m - q * P
            mul_r = jnp.where(r0 < 0, r0 + P, r0)
            opi = t - P
            res = jnp.where(opi == 0, add_r, jnp.where(opi == 1, sub_r, mul_r))
            res = jnp.where(t < P, t, jnp.where(t >= P + 3, t, res))
            outv[pl.ds(k * 16, 16)] = res
            packs.append(jnp.where(t < P, one, jnp.where(t < P + 3, one * 65536, 0)))

        pb = base + g * (16 * GRP)

        def fast(ops):
            acc, acc2, seg, nb = ops
            s = packs[0]
            for j in range(1, GRP):
                s = s + packs[j]
            return acc, acc2 + s, seg, nb

        def slow(ops):
            acc, acc2, seg, nb = ops
            acc = acc + jnp.where(lane == seg, jnp.sum(acc2), 0)
            for j in range(GRP):
                p0 = pb + j * 16
                lo = jnp.clip(cu_lo - p0, 0, 16)
                hi = jnp.clip(cu_hi - p0, 0, 16)
                ilo = jnp.maximum(lo - 1, 0)
                ihi = jnp.maximum(hi - 1, 0)
                cs = plsc.cumsum(packs[j])
                acc = acc + jnp.where(hi > 0, _take16(cs, ihi), 0) - \
                    jnp.where(lo > 0, _take16(cs, ilo), 0)
            pe = pb + 16 * GRP
            seg2 = jnp.sum(jnp.where(cu_hi <= pe, 1, 0))
            nb2 = jnp.min(jnp.where(cu_hi > pe, cu_hi, big))
            return acc, zeros, seg2, nb2

        return lax.cond(nb < pb + 16 * GRP, slow, fast, (acc, acc2, seg, nb))

    acc, acc2, seg, _ = lax.fori_loop(
        0, NVEC // GRP, body, (zeros, zeros, seg0, nb0))
    acc = acc + jnp.where(lane == seg, jnp.sum(acc2), 0)
    acc_o = jnp.bitwise_and(acc, 0xFFFF)
    acc_p = jnp.right_shift(acc, 16)

    cres = pltpu.make_async_copy(outv, res_hbm.at[pl.ds(base, CHUNK)], sem)
    cres.start()

    partial[0, :] = acc_o
    partial[1, :] = acc_p
    pltpu.sync_copy(partial, stage_hbm.at[sid])
    plsc.subcore_barrier()

    @pl.when(sid == 0)
    def _():
        pltpu.sync_copy(stage_hbm, red)
        ao = jnp.zeros((16,), jnp.int32)
        ap = jnp.zeros((16,), jnp.int32)
        for tix in range(NSUB):
            ao = ao + red[tix, 0, :]
            ap = ap + red[tix, 1, :]
        fo = ao.astype(jnp.float32)
        fp = ap.astype(jnp.float32)
        seg = lax.broadcasted_iota(jnp.int32, (16,), 0)
        plsc.store_scatter(cnt, [seg, zeros], fo)
        plsc.store_scatter(cnt, [seg, zeros + 1], fp)
        plsc.store_scatter(cnt, [seg, zeros + 2], fo + fp)
        pltpu.sync_copy(cnt, counts_hbm)

    cres.wait()


@jax.jit
def _run(tokens, cu_pad):
    mesh = plsc.VectorSubcoreMesh(
        core_axis_name="c", subcore_axis_name="s", num_cores=1)
    kern = pl.kernel(
        _sc_body,
        out_type=[
            jax.ShapeDtypeStruct((TOTAL,), jnp.int32),
            jax.ShapeDtypeStruct((16, 3), jnp.float32),
            jax.ShapeDtypeStruct((NSUB, 2, 16), jnp.int32),
        ],
        mesh=mesh,
        compiler_params=pltpu.CompilerParams(needs_layout_passes=False),
        scratch_types=[
            pltpu.VMEM((CHUNK + 32,), jnp.int32),
            pltpu.VMEM((24,), jnp.int32),
            pltpu.VMEM((CHUNK,), jnp.int32),
            pltpu.VMEM((2, 16), jnp.int32),
            pltpu.VMEM((NSUB, 2, 16), jnp.int32),
            pltpu.VMEM((16, 3), jnp.float32),
            pltpu.SemaphoreType.DMA,
        ],
    )
    return kern(tokens, cu_pad)


def kernel(tokens, cu_seqlens):
    res, counts, _ = _run(tokens, cu_seqlens)
    return res, counts


# speedup vs baseline: 1.0368x; 1.0084x over previous
import jax
import jax.numpy as jnp
from jax import lax
from jax.experimental import pallas as pl
from jax.experimental.pallas import tpu as pltpu
from jax.experimental.pallas import tpu_sc as plsc

P = 97
TOTAL = 32768
NSUB = 16
CHUNK = TOTAL // NSUB
NVEC = CHUNK // 16

_IN_BOUNDS = lax.GatherScatterMode.PROMISE_IN_BOUNDS


def _take16(v, idx):
    return v.at[idx].get(mode=_IN_BOUNDS)


def _mod_p(x):
    return jnp.where(x >= P, x - P, x)


def _subcore_id():
    return lax.axis_index("s")


def _sc_body(tokens_hbm, cu_hbm, res_hbm, counts_hbm, stage_hbm,
             chunk, cuv, outv, partial, red, cnt, sem):
    sid = _subcore_id()
    base = sid * CHUNK

    lh = jnp.where(sid == 0, TOTAL - 16, base - 16)
    rh = jnp.where(sid == NSUB - 1, 0, base + CHUNK)
    c0 = pltpu.make_async_copy(
        tokens_hbm.at[pl.ds(base, CHUNK)], chunk.at[pl.ds(16, CHUNK)], sem)
    c1 = pltpu.make_async_copy(
        tokens_hbm.at[pl.ds(lh, 16)], chunk.at[pl.ds(0, 16)], sem)
    c2 = pltpu.make_async_copy(
        tokens_hbm.at[pl.ds(rh, 16)], chunk.at[pl.ds(2048 + 16, 16)], sem)
    c3 = pltpu.make_async_copy(
        cu_hbm.at[pl.ds(0, 16)], cuv.at[pl.ds(0, 16)], sem)
    c0.start()
    c1.start()
    c2.start()
    c3.start()
    c0.wait()
    c1.wait()
    c2.wait()
    c3.wait()

    lane = lax.broadcasted_iota(jnp.int32, (16,), 0)
    cu_lo = cuv[pl.ds(0, 16)]
    cu_hi = jnp.where(lane == 15, TOTAL, cuv[pl.ds(1, 16)])
    zeros = jnp.zeros((16,), jnp.int32)

    def body(k, acc):
        off = 16 + k * 16
        t = chunk[pl.ds(off, 16)]
        tl = chunk[pl.ds(off - 1, 16)]
        tr = chunk[pl.ds(off + 1, 16)]

        l = _mod_p(tl)
        r = _mod_p(tr)
        add_r = _mod_p(l + r)
        d = l - r
        sub_r = jnp.where(d < 0, d + P, d)
        m = l * r
        q = jnp.right_shift(m * 676, 16)
        r0 = m - q * P
        mul_r = jnp.where(r0 < 0, r0 + P, r0)
        opi = t - P
        res = jnp.where(opi == 0, add_r, jnp.where(opi == 1, sub_r, mul_r))
        res = jnp.where(t < P, t, jnp.where(t >= P + 3, t, res))
        outv[pl.ds(k * 16, 16)] = res

        p0 = base + k * 16
        lo = jnp.clip(cu_lo - p0, 0, 16)
        hi = jnp.clip(cu_hi - p0, 0, 16)
        ilo = jnp.maximum(lo - 1, 0)
        ihi = jnp.maximum(hi - 1, 0)
        one = jnp.ones((16,), jnp.int32)
        packed = jnp.where(t < P, one, jnp.where(t < P + 3, one * 65536, 0))
        cs = plsc.cumsum(packed)
        diff = jnp.where(hi > 0, _take16(cs, ihi), 0) - \
            jnp.where(lo > 0, _take16(cs, ilo), 0)
        return acc + diff

    acc = lax.fori_loop(0, NVEC, body, zeros, unroll=8)
    acc_o = jnp.bitwise_and(acc, 0xFFFF)
    acc_p = jnp.right_shift(acc, 16)

    cres = pltpu.make_async_copy(outv, res_hbm.at[pl.ds(base, CHUNK)], sem)
    cres.start()

    partial[0, :] = acc_o
    partial[1, :] = acc_p
    pltpu.sync_copy(partial, stage_hbm.at[sid])
    plsc.subcore_barrier()

    @pl.when(sid == 0)
    def _():
        pltpu.sync_copy(stage_hbm, red)
        ao = jnp.zeros((16,), jnp.int32)
        ap = jnp.zeros((16,), jnp.int32)
        for tix in range(NSUB):
            ao = ao + red[tix, 0, :]
            ap = ap + red[tix, 1, :]
        fo = ao.astype(jnp.float32)
        fp = ap.astype(jnp.float32)
        seg = lax.broadcasted_iota(jnp.int32, (16,), 0)
        plsc.store_scatter(cnt, [seg, zeros], fo)
        plsc.store_scatter(cnt, [seg, zeros + 1], fp)
        plsc.store_scatter(cnt, [seg, zeros + 2], fo + fp)
        pltpu.sync_copy(cnt, counts_hbm)

    cres.wait()


@jax.jit
def _run(tokens, cu_pad):
    mesh = plsc.VectorSubcoreMesh(
        core_axis_name="c", subcore_axis_name="s", num_cores=1)
    kern = pl.kernel(
        _sc_body,
        out_type=[
            jax.ShapeDtypeStruct((TOTAL,), jnp.int32),
            jax.ShapeDtypeStruct((16, 3), jnp.float32),
            jax.ShapeDtypeStruct((NSUB, 2, 16), jnp.int32),
        ],
        mesh=mesh,
        compiler_params=pltpu.CompilerParams(needs_layout_passes=False),
        scratch_types=[
            pltpu.VMEM((CHUNK + 32,), jnp.int32),
            pltpu.VMEM((24,), jnp.int32),
            pltpu.VMEM((CHUNK,), jnp.int32),
            pltpu.VMEM((2, 16), jnp.int32),
            pltpu.VMEM((NSUB, 2, 16), jnp.int32),
            pltpu.VMEM((16, 3), jnp.float32),
            pltpu.SemaphoreType.DMA,
        ],
    )
    return kern(tokens, cu_pad)


def kernel(tokens, cu_seqlens):
    res, counts, _ = _run(tokens, cu_seqlens)
    return res, counts
